# baseline (speedup 1.0000x reference)
import functools

import jax
import jax.numpy as jnp
from jax import lax
from jax.experimental import pallas as pl
from jax.experimental.pallas import tpu as pltpu
from jax.experimental.pallas import tpu_sc as plsc

B = 4096
DIM = 1024
ND = 8

NC = 2
NS = 16
NW = NC * NS
BPW = B // NW
CH = 32
NCH = BPW // CH

T = 256
NT = B // T
MAXU = NT + ND - 1


def _wid():
    return lax.axis_index("s") * NC + lax.axis_index("c")


@functools.lru_cache(maxsize=None)
def _sc_kernels():
    mesh = plsc.VectorSubcoreMesh(core_axis_name="c", subcore_axis_name="s")

    @functools.partial(
        pl.kernel,
        mesh=mesh,
        out_type=jax.ShapeDtypeStruct((B, DIM), jnp.float32),
        scratch_types=[
            pltpu.VMEM((NCH, CH), jnp.int32),
            pltpu.VMEM((CH, DIM), jnp.float32),
            pltpu.SemaphoreType.DMA,
        ],
    )
    def sc_dispatch(mu_hbm, idx_hbm, o_hbm, idx_v, buf, sem):
        base = _wid() * BPW
        pltpu.sync_copy(idx_hbm.at[_wid()], idx_v)
        for ch in range(NCH):
            pltpu.sync_copy(mu_hbm.at[pl.ds(base + ch * CH, CH)], buf)
            pltpu.async_copy(buf, o_hbm.at[idx_v.at[ch]], sem).wait()

    @functools.partial(
        pl.kernel,
        mesh=mesh,
        out_type=jax.ShapeDtypeStruct((B, DIM), jnp.float32),
        scratch_types=[
            pltpu.VMEM((NCH, CH), jnp.int32),
            pltpu.VMEM((CH, DIM), jnp.float32),
            pltpu.SemaphoreType.DMA,
        ],
    )
    def sc_combine(ys_hbm, idx_hbm, o_hbm, idx_v, buf, sem):
        base = _wid() * BPW
        pltpu.sync_copy(idx_hbm.at[_wid()], idx_v)
        for ch in range(NCH):
            pltpu.async_copy(ys_hbm.at[idx_v.at[ch]], buf, sem).wait()
            pltpu.sync_copy(buf, o_hbm.at[pl.ds(base + ch * CH, CH)])

    return sc_dispatch, sc_combine


def _mm_body(um_ref, ug_ref, lo_ref, hi_ref, xs_ref, w_ref, o_ref, reg_ref,
             acc_ref, ssq_ref):
    u = pl.program_id(0)
    up = jnp.maximum(u - 1, 0)
    m = um_ref[u]
    first_m = jnp.logical_or(u == 0, um_ref[up] != m)
    first_g = jnp.logical_or(u == 0, ug_ref[up] != ug_ref[u])
    w = w_ref[0]

    @pl.when(u == 0)
    def _():
        acc_ref[...] = jnp.zeros_like(acc_ref)
        ssq_ref[0] = 0.0

    @pl.when(first_m)
    def _():
        o_ref[...] = jnp.zeros_like(o_ref)

    @pl.when(first_g)
    def _():
        acc_ref[...] += w
        ssq_ref[0] += jnp.sum(w * w)

    rows = m * T + lax.broadcasted_iota(jnp.int32, (T, 1), 0)
    mask = jnp.logical_and(rows >= lo_ref[u], rows < hi_ref[u])
    xw = lax.dot_general(
        xs_ref[...].astype(jnp.bfloat16), w.astype(jnp.bfloat16),
        (((1,), (1,)), ((), ())),
        preferred_element_type=jnp.float32,
    )
    o_ref[...] += jnp.where(mask, xw, 0.0)

    @pl.when(u == MAXU - 1)
    def _():
        a = acc_ref[...] * (1.0 / ND)
        reg_ref[0, 0] = ssq_ref[0] * (1.0 / (ND * DIM * DIM)) - jnp.sum(
            a * a) * (1.0 / (DIM * DIM))


def _grouped_matmul(um, ug, lo, hi, xs, W):
    grid_spec = pltpu.PrefetchScalarGridSpec(
        num_scalar_prefetch=4,
        grid=(MAXU,),
        in_specs=[
            pl.BlockSpec((T, DIM), lambda u, um, ug, lo, hi: (um[u], 0)),
            pl.BlockSpec((1, DIM, DIM), lambda u, um, ug, lo, hi: (ug[u], 0, 0)),
        ],
        out_specs=[
            pl.BlockSpec((T, DIM), lambda u, um, ug, lo, hi: (um[u], 0)),
            pl.BlockSpec((1, 1), lambda u, um, ug, lo, hi: (0, 0),
                         memory_space=pltpu.SMEM),
        ],
        scratch_shapes=[
            pltpu.VMEM((DIM, DIM), jnp.float32),
            pltpu.SMEM((1,), jnp.float32),
        ],
    )
    return pl.pallas_call(
        _mm_body,
        grid_spec=grid_spec,
        out_shape=[
            jax.ShapeDtypeStruct((B, DIM), jnp.float32),
            jax.ShapeDtypeStruct((1, 1), jnp.float32),
        ],
    )(um, ug, lo, hi, xs, W)


def _routing(ids):
    onehot = (jnp.arange(ND, dtype=jnp.int32)[:, None] == ids[None, :]).astype(
        jnp.int32)
    cum = jnp.cumsum(onehot, axis=1)
    counts = cum[:, -1]
    offs = jnp.concatenate([jnp.zeros((1,), jnp.int32),
                            jnp.cumsum(counts)]).astype(jnp.int32)
    rank = jnp.sum(onehot * cum, axis=0) - 1
    pos = offs[ids] + rank

    fg = jnp.minimum(offs[:-1] // T, NT - 1)
    lg = jnp.maximum(offs[1:] - 1, 0) // T
    ng = jnp.where(counts > 0, lg - fg + 1, 1)
    starts = jnp.concatenate([jnp.zeros((1,), jnp.int32),
                              jnp.cumsum(ng)[:-1]]).astype(jnp.int32)
    total = jnp.sum(ng)
    u = jnp.arange(MAXU, dtype=jnp.int32)
    valid = u < total
    uc = jnp.minimum(u, total - 1)
    g_of_u = (jnp.searchsorted(starts, uc, side="right") - 1).astype(jnp.int32)
    m_of_u = (fg[g_of_u] + (uc - starts[g_of_u])).astype(jnp.int32)
    lo = jnp.maximum(offs[g_of_u], m_of_u * T)
    hi = jnp.minimum(offs[g_of_u + 1], (m_of_u + 1) * T)
    lo = jnp.where(valid, lo, 0).astype(jnp.int32)
    hi = jnp.where(valid, hi, 0).astype(jnp.int32)
    return pos.astype(jnp.int32), m_of_u, g_of_u, lo, hi


def kernel(mu, domain_ids, W):
    ids = domain_ids.astype(jnp.int32)
    pos, um, ug, lo, hi = _routing(ids)
    idx3 = pos.reshape(NW, NCH, CH)

    sc_dispatch, sc_combine = _sc_kernels()
    xs = sc_dispatch(mu, idx3)
    ys, reg = _grouped_matmul(um, ug, lo, hi, xs, W)
    out = sc_combine(ys, idx3)
    return out, reg[0, 0]


# speedup vs baseline: 1.1602x; 1.1602x over previous
import functools

import jax
import jax.numpy as jnp
from jax import lax
from jax.experimental import pallas as pl
from jax.experimental.pallas import tpu as pltpu
from jax.experimental.pallas import tpu_sc as plsc

B = 4096
DIM = 1024
ND = 8

NC = 2
NS = 16
NW = NC * NS
BPW = B // NW
CH = 32
NCH = BPW // CH

T = 256
NT = B // T
MAXU = NT + ND - 1

RR = 32
RL = 128


def _wid():
    return lax.axis_index("s") * NC + lax.axis_index("c")


@functools.lru_cache(maxsize=None)
def _sc_kernels():
    mesh = plsc.VectorSubcoreMesh(core_axis_name="c", subcore_axis_name="s")

    @functools.partial(
        pl.kernel,
        mesh=mesh,
        out_type=jax.ShapeDtypeStruct((B, DIM), jnp.float32),
        scratch_types=[
            pltpu.VMEM((NCH, CH), jnp.int32),
            pltpu.VMEM((CH, DIM), jnp.float32),
            pltpu.SemaphoreType.DMA,
        ],
    )
    def sc_dispatch(mu_hbm, idx_hbm, o_hbm, idx_v, buf, sem):
        base = _wid() * BPW
        pltpu.sync_copy(idx_hbm.at[_wid()], idx_v)
        for ch in range(NCH):
            pltpu.sync_copy(mu_hbm.at[pl.ds(base + ch * CH, CH)], buf)
            pltpu.async_copy(buf, o_hbm.at[idx_v.at[ch]], sem).wait()

    @functools.partial(
        pl.kernel,
        mesh=mesh,
        out_type=jax.ShapeDtypeStruct((B, DIM), jnp.float32),
        scratch_types=[
            pltpu.VMEM((NCH, CH), jnp.int32),
            pltpu.VMEM((CH, DIM), jnp.float32),
            pltpu.SemaphoreType.DMA,
        ],
    )
    def sc_combine(ys_hbm, idx_hbm, o_hbm, idx_v, buf, sem):
        base = _wid() * BPW
        pltpu.sync_copy(idx_hbm.at[_wid()], idx_v)
        for ch in range(NCH):
            pltpu.async_copy(ys_hbm.at[idx_v.at[ch]], buf, sem).wait()
            pltpu.sync_copy(buf, o_hbm.at[pl.ds(base + ch * CH, CH)])

    return sc_dispatch, sc_combine


def _route_body(ids_ref, pos_ref, wk_ref):
    ids = ids_ref[...]
    tri_l = (lax.broadcasted_iota(jnp.int32, (RL, RL), 0) <=
             lax.broadcasted_iota(jnp.int32, (RL, RL), 1)).astype(jnp.float32)
    tri_r = (lax.broadcasted_iota(jnp.int32, (RR, RR), 1) <
             lax.broadcasted_iota(jnp.int32, (RR, RR), 0)).astype(jnp.float32)

    pos = jnp.zeros((RR, RL), jnp.float32)
    off = jnp.float32(0.0)
    offs = []
    for d in range(ND):
        offs.append(off)
        eq = (ids == d).astype(jnp.float32)
        prefix = lax.dot_general(eq, tri_l, (((1,), (0,)), ((), ())),
                                 preferred_element_type=jnp.float32)
        t = jnp.sum(eq, axis=1, keepdims=True)
        before = lax.dot_general(tri_r, t, (((1,), (0,)), ((), ())),
                                 preferred_element_type=jnp.float32)
        rank = before + prefix - eq
        pos = pos + eq * (off + rank)
        off = off + jnp.sum(t)
    offs.append(off)
    pos_ref[...] = pos.astype(jnp.int32)

    ioffs = [o.astype(jnp.int32) for o in offs]
    u = lax.broadcasted_iota(jnp.int32, (1, RL), 1)
    starts_g = []
    start = jnp.int32(0)
    fg_l, ng_l = [], []
    for g in range(ND):
        cnt = ioffs[g + 1] - ioffs[g]
        fg = jnp.minimum(ioffs[g] // T, NT - 1)
        lg = jnp.maximum(ioffs[g + 1] - 1, 0) // T
        ng = jnp.where(cnt > 0, lg - fg + 1, 1)
        starts_g.append(start)
        fg_l.append(fg)
        ng_l.append(ng)
        start = start + ng
    total = start
    uc = jnp.minimum(u, total - 1)
    g_of = jnp.zeros((1, RL), jnp.int32)
    for g in range(ND):
        g_of = g_of + (starts_g[g] <= uc).astype(jnp.int32)
    g_of = g_of - 1
    m_of = jnp.zeros((1, RL), jnp.int32)
    lo = jnp.zeros((1, RL), jnp.int32)
    hi = jnp.zeros((1, RL), jnp.int32)
    for g in range(ND):
        sel = (g_of == g)
        m_g = fg_l[g] + (uc - starts_g[g])
        m_of = jnp.where(sel, m_g, m_of)
        lo = jnp.where(sel, jnp.maximum(ioffs[g], m_g * T), lo)
        hi = jnp.where(sel, jnp.minimum(ioffs[g + 1], (m_g + 1) * T), hi)
    valid = u < total
    lo = jnp.where(valid, lo, 0)
    hi = jnp.where(valid, hi, 0)
    wk_ref[0:1, :] = m_of
    wk_ref[1:2, :] = g_of
    wk_ref[2:3, :] = lo
    wk_ref[3:4, :] = hi


def _routing(ids2):
    return pl.pallas_call(
        _route_body,
        out_shape=[
            jax.ShapeDtypeStruct((RR, RL), jnp.int32),
            jax.ShapeDtypeStruct((4, RL), jnp.int32),
        ],
    )(ids2)


def _mm_body(wk_ref, xs_ref, w_ref, o_ref):
    u = pl.program_id(0)
    up = jnp.maximum(u - 1, 0)
    m = wk_ref[0, u]
    first_m = jnp.logical_or(u == 0, wk_ref[0, up] != m)

    @pl.when(first_m)
    def _():
        o_ref[...] = jnp.zeros_like(o_ref)

    rows = m * T + lax.broadcasted_iota(jnp.int32, (T, 1), 0)
    mask = jnp.logical_and(rows >= wk_ref[2, u], rows < wk_ref[3, u])
    xw = lax.dot_general(
        xs_ref[...], w_ref[0],
        (((1,), (1,)), ((), ())),
        preferred_element_type=jnp.float32,
    )
    o_ref[...] += jnp.where(mask, xw, 0.0)


def _grouped_matmul(wk, xs, W):
    grid_spec = pltpu.PrefetchScalarGridSpec(
        num_scalar_prefetch=1,
        grid=(MAXU,),
        in_specs=[
            pl.BlockSpec((T, DIM), lambda u, wk: (wk[0, u], 0)),
            pl.BlockSpec((1, DIM, DIM), lambda u, wk: (wk[1, u], 0, 0)),
        ],
        out_specs=pl.BlockSpec((T, DIM), lambda u, wk: (wk[0, u], 0)),
    )
    return pl.pallas_call(
        _mm_body,
        grid_spec=grid_spec,
        out_shape=jax.ShapeDtypeStruct((B, DIM), jnp.float32),
    )(wk, xs, W)


def _reg_body(w_ref, o_ref, acc_ref, ssq_ref):
    i = pl.program_id(0)
    w = w_ref[0]

    @pl.when(i == 0)
    def _():
        acc_ref[...] = jnp.zeros_like(acc_ref)
        ssq_ref[0] = 0.0

    acc_ref[...] += w
    ssq_ref[0] += jnp.sum(w * w)

    @pl.when(i == ND - 1)
    def _():
        a = acc_ref[...] * (1.0 / ND)
        o_ref[0, 0] = ssq_ref[0] * (1.0 / (ND * DIM * DIM)) - jnp.sum(
            a * a) * (1.0 / (DIM * DIM))


def _reg_loss(W):
    return pl.pallas_call(
        _reg_body,
        grid=(ND,),
        in_specs=[pl.BlockSpec((1, DIM, DIM), lambda i: (i, 0, 0))],
        out_specs=pl.BlockSpec((1, 1), lambda i: (0, 0), memory_space=pltpu.SMEM),
        out_shape=jax.ShapeDtypeStruct((1, 1), jnp.float32),
        scratch_shapes=[
            pltpu.VMEM((DIM, DIM), jnp.float32),
            pltpu.SMEM((1,), jnp.float32),
        ],
    )(W)


def kernel(mu, domain_ids, W):
    ids2 = domain_ids.astype(jnp.int32).reshape(RR, RL)
    reg = _reg_loss(W)
    pos, wk = _routing(ids2)
    idx3 = pos.reshape(NW, NCH, CH)

    sc_dispatch, sc_combine = _sc_kernels()
    xs = sc_dispatch(mu, idx3)
    ys = _grouped_matmul(wk, xs, W)
    out = sc_combine(ys, idx3)
    return out, reg[0, 0]


# speedup vs baseline: 1.1693x; 1.0079x over previous
import functools

import jax
import jax.numpy as jnp
from jax import lax
from jax.experimental import pallas as pl
from jax.experimental.pallas import tpu as pltpu
from jax.experimental.pallas import tpu_sc as plsc

B = 4096
DIM = 1024
ND = 8

NC = 2
NS = 16
NW = NC * NS
BPW = B // NW
CH = 32
NCH = BPW // CH

T = 256
NT = B // T
MAXU = NT + ND - 1

RR = 32
RL = 128


def _wid():
    return lax.axis_index("s") * NC + lax.axis_index("c")


@functools.lru_cache(maxsize=None)
def _sc_kernels():
    mesh = plsc.VectorSubcoreMesh(core_axis_name="c", subcore_axis_name="s")

    sc_scratch = [
        pltpu.VMEM((NCH, CH), jnp.int32),
        pltpu.VMEM((CH, DIM), jnp.float32),
        pltpu.VMEM((CH, DIM), jnp.float32),
        pltpu.SemaphoreType.DMA,
        pltpu.SemaphoreType.DMA,
        pltpu.SemaphoreType.DMA,
        pltpu.SemaphoreType.DMA,
    ]

    @functools.partial(
        pl.kernel,
        mesh=mesh,
        out_type=jax.ShapeDtypeStruct((B, DIM), jnp.float32),
        scratch_types=sc_scratch,
    )
    def sc_dispatch(mu_hbm, idx_hbm, o_hbm, idx_v, buf0, buf1, si0, si1, so0, so1):
        base = _wid() * BPW
        pltpu.sync_copy(idx_hbm.at[_wid()], idx_v)
        bufs, sin, sout = (buf0, buf1), (si0, si1), (so0, so1)
        cp_in, cp_out = {}, {}
        for ch in range(min(2, NCH)):
            cp_in[ch] = pltpu.async_copy(
                mu_hbm.at[pl.ds(base + ch * CH, CH)], bufs[ch % 2], sin[ch % 2])
        for ch in range(NCH):
            b = ch % 2
            cp_in[ch].wait()
            cp_out[ch] = pltpu.async_copy(bufs[b], o_hbm.at[idx_v.at[ch]], sout[b])
            if ch + 2 < NCH:
                cp_out[ch].wait()
                cp_in[ch + 2] = pltpu.async_copy(
                    mu_hbm.at[pl.ds(base + (ch + 2) * CH, CH)], bufs[b], sin[b])
        for ch in range(max(0, NCH - 2), NCH):
            cp_out[ch].wait()

    @functools.partial(
        pl.kernel,
        mesh=mesh,
        out_type=jax.ShapeDtypeStruct((B, DIM), jnp.float32),
        scratch_types=sc_scratch,
    )
    def sc_combine(ys_hbm, idx_hbm, o_hbm, idx_v, buf0, buf1, si0, si1, so0, so1):
        base = _wid() * BPW
        pltpu.sync_copy(idx_hbm.at[_wid()], idx_v)
        bufs, sin, sout = (buf0, buf1), (si0, si1), (so0, so1)
        cp_in, cp_out = {}, {}
        for ch in range(min(2, NCH)):
            cp_in[ch] = pltpu.async_copy(
                ys_hbm.at[idx_v.at[ch]], bufs[ch % 2], sin[ch % 2])
        for ch in range(NCH):
            b = ch % 2
            cp_in[ch].wait()
            cp_out[ch] = pltpu.async_copy(
                bufs[b], o_hbm.at[pl.ds(base + ch * CH, CH)], sout[b])
            if ch + 2 < NCH:
                cp_out[ch].wait()
                cp_in[ch + 2] = pltpu.async_copy(
                    ys_hbm.at[idx_v.at[ch + 2]], bufs[b], sin[b])
        for ch in range(max(0, NCH - 2), NCH):
            cp_out[ch].wait()

    return sc_dispatch, sc_combine


def _route_body(ids_ref, pos_ref, wk_ref):
    ids = ids_ref[...]
    tri_l = (lax.broadcasted_iota(jnp.int32, (RL, RL), 0) <=
             lax.broadcasted_iota(jnp.int32, (RL, RL), 1)).astype(jnp.float32)
    tri_r = (lax.broadcasted_iota(jnp.int32, (RR, RR), 1) <
             lax.broadcasted_iota(jnp.int32, (RR, RR), 0)).astype(jnp.float32)

    pos = jnp.zeros((RR, RL), jnp.float32)
    off = jnp.float32(0.0)
    offs = []
    for d in range(ND):
        offs.append(off)
        eq = (ids == d).astype(jnp.float32)
        prefix = lax.dot_general(eq, tri_l, (((1,), (0,)), ((), ())),
                                 preferred_element_type=jnp.float32)
        t = jnp.sum(eq, axis=1, keepdims=True)
        before = lax.dot_general(tri_r, t, (((1,), (0,)), ((), ())),
                                 preferred_element_type=jnp.float32)
        rank = before + prefix - eq
        pos = pos + eq * (off + rank)
        off = off + jnp.sum(t)
    offs.append(off)
    pos_ref[...] = pos.astype(jnp.int32)

    ioffs = [o.astype(jnp.int32) for o in offs]
    u = lax.broadcasted_iota(jnp.int32, (1, RL), 1)
    starts_g = []
    start = jnp.int32(0)
    fg_l, ng_l = [], []
    for g in range(ND):
        cnt = ioffs[g + 1] - ioffs[g]
        fg = jnp.minimum(ioffs[g] // T, NT - 1)
        lg = jnp.maximum(ioffs[g + 1] - 1, 0) // T
        ng = jnp.where(cnt > 0, lg - fg + 1, 1)
        starts_g.append(start)
        fg_l.append(fg)
        ng_l.append(ng)
        start = start + ng
    total = start
    uc = jnp.minimum(u, total - 1)
    g_of = jnp.zeros((1, RL), jnp.int32)
    for g in range(ND):
        g_of = g_of + (starts_g[g] <= uc).astype(jnp.int32)
    g_of = g_of - 1
    m_of = jnp.zeros((1, RL), jnp.int32)
    lo = jnp.zeros((1, RL), jnp.int32)
    hi = jnp.zeros((1, RL), jnp.int32)
    for g in range(ND):
        sel = (g_of == g)
        m_g = fg_l[g] + (uc - starts_g[g])
        m_of = jnp.where(sel, m_g, m_of)
        lo = jnp.where(sel, jnp.maximum(ioffs[g], m_g * T), lo)
        hi = jnp.where(sel, jnp.minimum(ioffs[g + 1], (m_g + 1) * T), hi)
    valid = u < total
    lo = jnp.where(valid, lo, 0)
    hi = jnp.where(valid, hi, 0)
    wk_ref[0:1, :] = m_of
    wk_ref[1:2, :] = g_of
    wk_ref[2:3, :] = lo
    wk_ref[3:4, :] = hi


def _routing(ids2):
    return pl.pallas_call(
        _route_body,
        out_shape=[
            jax.ShapeDtypeStruct((RR, RL), jnp.int32),
            jax.ShapeDtypeStruct((4, RL), jnp.int32),
        ],
    )(ids2)


def _mm_body(wk_ref, xs_ref, w_ref, o_ref):
    u = pl.program_id(0)
    up = jnp.maximum(u - 1, 0)
    m = wk_ref[0, u]
    first_m = jnp.logical_or(u == 0, wk_ref[0, up] != m)

    @pl.when(first_m)
    def _():
        o_ref[...] = jnp.zeros_like(o_ref)

    rows = m * T + lax.broadcasted_iota(jnp.int32, (T, 1), 0)
    mask = jnp.logical_and(rows >= wk_ref[2, u], rows < wk_ref[3, u])
    xw = lax.dot_general(
        xs_ref[...], w_ref[0],
        (((1,), (1,)), ((), ())),
        preferred_element_type=jnp.float32,
    )
    o_ref[...] += jnp.where(mask, xw, 0.0)


def _grouped_matmul(wk, xs, W):
    grid_spec = pltpu.PrefetchScalarGridSpec(
        num_scalar_prefetch=1,
        grid=(MAXU,),
        in_specs=[
            pl.BlockSpec((T, DIM), lambda u, wk: (wk[0, u], 0)),
            pl.BlockSpec((1, DIM, DIM), lambda u, wk: (wk[1, u], 0, 0)),
        ],
        out_specs=pl.BlockSpec((T, DIM), lambda u, wk: (wk[0, u], 0)),
    )
    return pl.pallas_call(
        _mm_body,
        grid_spec=grid_spec,
        out_shape=jax.ShapeDtypeStruct((B, DIM), jnp.float32),
    )(wk, xs, W)


def _reg_body(w_ref, dep_ref, o_ref, acc_ref, ssq_ref):
    del dep_ref
    i = pl.program_id(0)
    w = w_ref[0]

    @pl.when(i == 0)
    def _():
        acc_ref[...] = jnp.zeros_like(acc_ref)
        ssq_ref[0] = 0.0

    acc_ref[...] += w
    ssq_ref[0] += jnp.sum(w * w)

    @pl.when(i == ND - 1)
    def _():
        a = acc_ref[...] * (1.0 / ND)
        o_ref[0, 0] = ssq_ref[0] * (1.0 / (ND * DIM * DIM)) - jnp.sum(
            a * a) * (1.0 / (DIM * DIM))


def _reg_loss(W, dep):
    return pl.pallas_call(
        _reg_body,
        grid=(ND,),
        in_specs=[
            pl.BlockSpec((1, DIM, DIM), lambda i: (i, 0, 0)),
            pl.BlockSpec((8, 128), lambda i: (0, 0)),
        ],
        out_specs=pl.BlockSpec((1, 1), lambda i: (0, 0), memory_space=pltpu.SMEM),
        out_shape=jax.ShapeDtypeStruct((1, 1), jnp.float32),
        scratch_shapes=[
            pltpu.VMEM((DIM, DIM), jnp.float32),
            pltpu.SMEM((1,), jnp.float32),
        ],
    )(W, dep)


def kernel(mu, domain_ids, W):
    ids2 = domain_ids.astype(jnp.int32).reshape(RR, RL)
    pos, wk = _routing(ids2)
    idx3 = pos.reshape(NW, NCH, CH)

    sc_dispatch, sc_combine = _sc_kernels()
    xs = sc_dispatch(mu, idx3)
    ys = _grouped_matmul(wk, xs, W)
    out = sc_combine(ys, idx3)
    reg = _reg_loss(W, ys)
    return out, reg[0, 0]
